# baseline (speedup 1.0000x reference)
import jax
import jax.numpy as jnp
from jax import lax
from jax.experimental import pallas as pl
from jax.experimental.pallas import tpu as pltpu
from jax.experimental.pallas import tpu_sc as plsc

H, W = 480, 640
HW = H * W
HHW = HW // 2
NC, NS, L = 2, 16, 16
CHUNK = HW // NS
RPX = 640
NR = CHUNK // RPX
NG = RPX // L
GPAD = 2048
ACC_ROWS = HHW + GPAD
ZSL = ACC_ROWS // NS
OSL = HHW // NS


def _sc_body(scale_hbm, fx_hbm, fy_hbm, zeros_hbm, oy_hbm, ox_hbm,
             acc_w, acc_y, acc_x, scale_v, fx_v, fy_v,
             idx0, idx1, idx2, idx3, dat0, dat1, dat2, dat3,
             wb_w, wb_y, wb_x, sem):
    cid = lax.axis_index("c")
    sid = lax.axis_index("s")
    base = sid * CHUNK
    half0 = cid * HHW

    pltpu.sync_copy(scale_hbm, scale_v)
    pltpu.sync_copy(fx_hbm.at[pl.ds(base, CHUNK)], fx_v)
    pltpu.sync_copy(fy_hbm.at[pl.ds(base, CHUNK)], fy_v)

    pltpu.sync_copy(zeros_hbm, wb_w)
    for acc in (acc_w, acc_y, acc_x):
        pltpu.sync_copy(wb_w.at[pl.ds(0, ZSL)], acc.at[pl.ds(sid * ZSL, ZSL)])
    plsc.subcore_barrier()

    sc = scale_v[...]
    lanes = lax.iota(jnp.int32, 16)
    fzero = jnp.zeros((16,), jnp.float32)
    fone = jnp.ones((16,), jnp.float32)

    rows_per_round = W // RPX
    base_row = sid * (CHUNK // W)

    def round_body(r, carry):
        anys = [None, None, None, None]
        yrow = base_row + r // rows_per_round
        x0 = (r - (r // rows_per_round) * rows_per_round) * RPX
        yrow_f = yrow.astype(jnp.float32)
        for g in range(NG):
            off = r * RPX + g * L
            q = off + lanes
            xi = x0 + g * L + lanes
            fyv = fy_v[pl.ds(off, L)]
            fxv = fx_v[pl.ds(off, L)]
            wy = yrow_f + sc * fyv
            wx = xi.astype(jnp.float32) + sc * fxv
            inb = ((wy >= 0.0) & (wy < float(H))
                   & (wx >= 0.0) & (wx < float(W)))
            inbf = jnp.where(inb, fone, fzero)
            wy = wy * inbf
            wx = wx * inbf
            t = wy.astype(jnp.int32)
            l = wx.astype(jnp.int32)
            dy = wy - t.astype(jnp.float32)
            dx = wx - l.astype(jnp.float32)
            bok = (t + 1) < H
            rok = (l + 1) < W
            bf = jnp.where(bok, fone, fzero)
            rf = jnp.where(rok, fone, fzero)
            omdy = 1.0 - dy
            omdx = 1.0 - dx
            w_tl = omdy * omdx * inbf
            w_tr = omdy * dx * rf * inbf
            w_bl = dy * omdx * bf * inbf
            w_br = dy * dx * rf * bf * inbf
            tl = t * W + l
            garbage = HHW + (q & (GPAD - 1))
            for ci, (idx_ref, dat_ref, ok_c, i_c, w_c) in enumerate((
                    (idx0, dat0, inb, tl, w_tl),
                    (idx1, dat1, inb & rok, tl + 1, w_tr),
                    (idx2, dat2, inb & bok, tl + W, w_bl),
                    (idx3, dat3, inb & rok & bok, tl + W + 1, w_br))):
                local = ok_c & (i_c >= half0) & (i_c < half0 + HHW)
                idx_ref[pl.ds(g * L, L)] = jnp.where(local, i_c - half0,
                                                     garbage)
                dat_ref[pl.ds(g * L, L)] = w_c
                dat_ref[pl.ds(RPX + g * L, L)] = w_c * fyv
                dat_ref[pl.ds(2 * RPX + g * L, L)] = w_c * fxv
                any_g = jnp.any(local)
                anys[ci] = any_g if g == 0 else (anys[ci] | any_g)

        for any_c, idx_ref, dat_ref in ((anys[0], idx0, dat0),
                                        (anys[1], idx1, dat1),
                                        (anys[2], idx2, dat2),
                                        (anys[3], idx3, dat3)):
            @pl.when(any_c)
            def _(idx_ref=idx_ref, dat_ref=dat_ref):
                d0 = pltpu.async_copy(dat_ref.at[pl.ds(0, RPX)],
                                      acc_w.at[idx_ref], sem, add=True)
                d1 = pltpu.async_copy(dat_ref.at[pl.ds(RPX, RPX)],
                                      acc_y.at[idx_ref], sem, add=True)
                d2 = pltpu.async_copy(dat_ref.at[pl.ds(2 * RPX, RPX)],
                                      acc_x.at[idx_ref], sem, add=True)
                d0.wait()
                d1.wait()
                d2.wait()

        return carry

    lax.fori_loop(0, NR, round_body, 0)
    plsc.subcore_barrier()

    o = sid * OSL
    pltpu.sync_copy(acc_w.at[pl.ds(o, OSL)], wb_w.at[pl.ds(0, OSL)])
    pltpu.sync_copy(acc_y.at[pl.ds(o, OSL)], wb_y)
    pltpu.sync_copy(acc_x.at[pl.ds(o, OSL)], wb_x)

    def div_body(i, carry):
        s = pl.ds(i * L, L)
        wsum = wb_w[s] + 1e-9
        wb_y[s] = wb_y[s] / wsum
        wb_x[s] = wb_x[s] / wsum
        return carry

    lax.fori_loop(0, OSL // L, div_body, 0)
    gbase = cid * HHW + o
    pltpu.sync_copy(wb_y, oy_hbm.at[pl.ds(gbase, OSL)])
    pltpu.sync_copy(wb_x, ox_hbm.at[pl.ds(gbase, OSL)])


_mesh = plsc.VectorSubcoreMesh(core_axis_name="c", subcore_axis_name="s")
_params = pltpu.CompilerParams(needs_layout_passes=False,
                               use_tc_tiling_on_sc=False)

_sc_call = pl.kernel(
    _sc_body,
    out_type=(jax.ShapeDtypeStruct((HW,), jnp.float32),
              jax.ShapeDtypeStruct((HW,), jnp.float32)),
    mesh=_mesh,
    compiler_params=_params,
    scratch_types=[
        pltpu.VMEM_SHARED((ACC_ROWS,), jnp.float32),
        pltpu.VMEM_SHARED((ACC_ROWS,), jnp.float32),
        pltpu.VMEM_SHARED((ACC_ROWS,), jnp.float32),
        pltpu.VMEM((16,), jnp.float32),
        pltpu.VMEM((CHUNK,), jnp.float32),
        pltpu.VMEM((CHUNK,), jnp.float32),
        pltpu.VMEM((RPX,), jnp.int32),
        pltpu.VMEM((RPX,), jnp.int32),
        pltpu.VMEM((RPX,), jnp.int32),
        pltpu.VMEM((RPX,), jnp.int32),
        pltpu.VMEM((3 * RPX,), jnp.float32),
        pltpu.VMEM((3 * RPX,), jnp.float32),
        pltpu.VMEM((3 * RPX,), jnp.float32),
        pltpu.VMEM((3 * RPX,), jnp.float32),
        pltpu.VMEM((ZSL,), jnp.float32),
        pltpu.VMEM((OSL,), jnp.float32),
        pltpu.VMEM((OSL,), jnp.float32),
        pltpu.SemaphoreType.DMA,
    ],
)


@jax.jit
def kernel(i, tref, flow_maps_x, flow_maps_y):
    fx = lax.dynamic_index_in_dim(flow_maps_x[0], i, axis=0, keepdims=False)
    fy = lax.dynamic_index_in_dim(flow_maps_y[0], i, axis=0, keepdims=False)
    fx_flat = fx.reshape(HW)
    fy_flat = fy.reshape(HW)
    scale = jnp.asarray(tref - i, jnp.float32)
    scale_arr = jnp.broadcast_to(scale, (16,))
    zeros = jnp.zeros((ZSL,), jnp.float32)
    oy, ox = _sc_call(scale_arr, fx_flat, fy_flat, zeros)
    return (ox.reshape(1, 1, H, W), oy.reshape(1, 1, H, W))


# speedup vs baseline: 1.1256x; 1.1256x over previous
import jax
import jax.numpy as jnp
from jax import lax
from jax.experimental import pallas as pl
from jax.experimental.pallas import tpu as pltpu
from jax.experimental.pallas import tpu_sc as plsc

H, W = 480, 640
HW = H * W
HHW = HW // 2
NC, NS, L = 2, 16, 16
CHUNK = HW // NS
RPX = 128
NR = CHUNK // RPX
NG = RPX // L
GPAD = 2048
ACC_ROWS = HHW + GPAD
ZSL = ACC_ROWS // NS
OSL = HHW // NS


def _sc_body(scale_hbm, fx_hbm, fy_hbm, zeros_hbm, oy_hbm, ox_hbm,
             acc_w, acc_y, acc_x, scale_v, fx_v, fy_v,
             idx0, idx1, idx2, idx3, dat0, dat1, dat2, dat3,
             wb_w, wb_y, wb_x, sem):
    cid = lax.axis_index("c")
    sid = lax.axis_index("s")
    half0 = cid * HHW

    HB = CHUNK // 2
    baseA = sid * HB
    baseB = HHW + baseA

    pltpu.sync_copy(scale_hbm, scale_v)
    pltpu.sync_copy(fx_hbm.at[pl.ds(baseA, HB)], fx_v.at[pl.ds(0, HB)])
    pltpu.sync_copy(fx_hbm.at[pl.ds(baseB, HB)], fx_v.at[pl.ds(HB, HB)])
    pltpu.sync_copy(fy_hbm.at[pl.ds(baseA, HB)], fy_v.at[pl.ds(0, HB)])
    pltpu.sync_copy(fy_hbm.at[pl.ds(baseB, HB)], fy_v.at[pl.ds(HB, HB)])

    pltpu.sync_copy(zeros_hbm, wb_w)
    for acc in (acc_w, acc_y, acc_x):
        pltpu.sync_copy(wb_w.at[pl.ds(0, ZSL)], acc.at[pl.ds(sid * ZSL, ZSL)])
    plsc.subcore_barrier()

    sc = scale_v[...]
    lanes = lax.iota(jnp.int32, 16)
    fzero = jnp.zeros((16,), jnp.float32)
    fone = jnp.ones((16,), jnp.float32)

    rows_per_round = W // RPX
    rowsA = sid * (HB // W)
    NRH = NR // 2

    def round_body(r, carry):
        anys = [None, None, None, None]
        rr = jnp.where(r < NRH, r, r - NRH)
        row_in_block = rr // rows_per_round
        yrow = rowsA + row_in_block + jnp.where(r < NRH, 0, H // 2)
        x0 = (rr - row_in_block * rows_per_round) * RPX
        yrow_f = yrow.astype(jnp.float32)
        for g in range(NG):
            off = r * RPX + g * L
            q = off + lanes
            xi = x0 + g * L + lanes
            fyv = fy_v[pl.ds(off, L)]
            fxv = fx_v[pl.ds(off, L)]
            wy = yrow_f + sc * fyv
            wx = xi.astype(jnp.float32) + sc * fxv
            inb = ((wy >= 0.0) & (wy < float(H))
                   & (wx >= 0.0) & (wx < float(W)))
            inbf = jnp.where(inb, fone, fzero)
            wy = wy * inbf
            wx = wx * inbf
            t = wy.astype(jnp.int32)
            l = wx.astype(jnp.int32)
            dy = wy - t.astype(jnp.float32)
            dx = wx - l.astype(jnp.float32)
            bok = (t + 1) < H
            rok = (l + 1) < W
            bf = jnp.where(bok, fone, fzero)
            rf = jnp.where(rok, fone, fzero)
            omdy = 1.0 - dy
            omdx = 1.0 - dx
            w_tl = omdy * omdx * inbf
            w_tr = omdy * dx * rf * inbf
            w_bl = dy * omdx * bf * inbf
            w_br = dy * dx * rf * bf * inbf
            tl = t * W + l
            garbage = HHW + (q & (GPAD - 1))
            for ci, (idx_ref, dat_ref, ok_c, i_c, w_c) in enumerate((
                    (idx0, dat0, inb, tl, w_tl),
                    (idx1, dat1, inb & rok, tl + 1, w_tr),
                    (idx2, dat2, inb & bok, tl + W, w_bl),
                    (idx3, dat3, inb & rok & bok, tl + W + 1, w_br))):
                local = ok_c & (i_c >= half0) & (i_c < half0 + HHW)
                idx_ref[pl.ds(g * L, L)] = jnp.where(local, i_c - half0,
                                                     garbage)
                dat_ref[pl.ds(g * L, L)] = w_c
                dat_ref[pl.ds(RPX + g * L, L)] = w_c * fyv
                dat_ref[pl.ds(2 * RPX + g * L, L)] = w_c * fxv
                any_g = jnp.any(local)
                anys[ci] = any_g if g == 0 else (anys[ci] | any_g)

        for any_c, idx_ref, dat_ref in ((anys[0], idx0, dat0),
                                        (anys[1], idx1, dat1),
                                        (anys[2], idx2, dat2),
                                        (anys[3], idx3, dat3)):
            @pl.when(any_c)
            def _(idx_ref=idx_ref, dat_ref=dat_ref):
                d0 = pltpu.async_copy(dat_ref.at[pl.ds(0, RPX)],
                                      acc_w.at[idx_ref], sem, add=True)
                d1 = pltpu.async_copy(dat_ref.at[pl.ds(RPX, RPX)],
                                      acc_y.at[idx_ref], sem, add=True)
                d2 = pltpu.async_copy(dat_ref.at[pl.ds(2 * RPX, RPX)],
                                      acc_x.at[idx_ref], sem, add=True)
                d0.wait()
                d1.wait()
                d2.wait()

        return carry

    lax.fori_loop(0, NR, round_body, 0)
    plsc.subcore_barrier()

    o = sid * OSL
    pltpu.sync_copy(acc_w.at[pl.ds(o, OSL)], wb_w.at[pl.ds(0, OSL)])
    pltpu.sync_copy(acc_y.at[pl.ds(o, OSL)], wb_y)
    pltpu.sync_copy(acc_x.at[pl.ds(o, OSL)], wb_x)

    def div_body(i, carry):
        s = pl.ds(i * L, L)
        wsum = wb_w[s] + 1e-9
        wb_y[s] = wb_y[s] / wsum
        wb_x[s] = wb_x[s] / wsum
        return carry

    lax.fori_loop(0, OSL // L, div_body, 0)
    gbase = cid * HHW + o
    pltpu.sync_copy(wb_y, oy_hbm.at[pl.ds(gbase, OSL)])
    pltpu.sync_copy(wb_x, ox_hbm.at[pl.ds(gbase, OSL)])


_mesh = plsc.VectorSubcoreMesh(core_axis_name="c", subcore_axis_name="s")
_params = pltpu.CompilerParams(needs_layout_passes=False,
                               use_tc_tiling_on_sc=False)

_sc_call = pl.kernel(
    _sc_body,
    out_type=(jax.ShapeDtypeStruct((HW,), jnp.float32),
              jax.ShapeDtypeStruct((HW,), jnp.float32)),
    mesh=_mesh,
    compiler_params=_params,
    scratch_types=[
        pltpu.VMEM_SHARED((ACC_ROWS,), jnp.float32),
        pltpu.VMEM_SHARED((ACC_ROWS,), jnp.float32),
        pltpu.VMEM_SHARED((ACC_ROWS,), jnp.float32),
        pltpu.VMEM((16,), jnp.float32),
        pltpu.VMEM((CHUNK,), jnp.float32),
        pltpu.VMEM((CHUNK,), jnp.float32),
        pltpu.VMEM((RPX,), jnp.int32),
        pltpu.VMEM((RPX,), jnp.int32),
        pltpu.VMEM((RPX,), jnp.int32),
        pltpu.VMEM((RPX,), jnp.int32),
        pltpu.VMEM((3 * RPX,), jnp.float32),
        pltpu.VMEM((3 * RPX,), jnp.float32),
        pltpu.VMEM((3 * RPX,), jnp.float32),
        pltpu.VMEM((3 * RPX,), jnp.float32),
        pltpu.VMEM((ZSL,), jnp.float32),
        pltpu.VMEM((OSL,), jnp.float32),
        pltpu.VMEM((OSL,), jnp.float32),
        pltpu.SemaphoreType.DMA,
    ],
)


@jax.jit
def kernel(i, tref, flow_maps_x, flow_maps_y):
    fx = lax.dynamic_index_in_dim(flow_maps_x[0], i, axis=0, keepdims=False)
    fy = lax.dynamic_index_in_dim(flow_maps_y[0], i, axis=0, keepdims=False)
    fx_flat = fx.reshape(HW)
    fy_flat = fy.reshape(HW)
    scale = jnp.asarray(tref - i, jnp.float32)
    scale_arr = jnp.broadcast_to(scale, (16,))
    zeros = jnp.zeros((ZSL,), jnp.float32)
    oy, ox = _sc_call(scale_arr, fx_flat, fy_flat, zeros)
    return (ox.reshape(1, 1, H, W), oy.reshape(1, 1, H, W))


# speedup vs baseline: 1.3194x; 1.1722x over previous
import jax
import jax.numpy as jnp
from jax import lax
from jax.experimental import pallas as pl
from jax.experimental.pallas import tpu as pltpu
from jax.experimental.pallas import tpu_sc as plsc

H, W = 480, 640
HW = H * W
HHW = HW // 2
NC, NS, L = 2, 16, 16
CHUNK = HW // NS
RPX = 128
NR = CHUNK // RPX
NG = RPX // L
GPAD = 2048
ACC_ROWS = HHW + GPAD
ZSL = ACC_ROWS // NS
OSL = HHW // NS


def _sc_body(scale_hbm, fx_hbm, fy_hbm, zeros_hbm, oy_hbm, ox_hbm,
             acc_w, acc_y, acc_x, scale_v, fx_v, fy_v,
             idx0, idx1, idx2, idx3, dat0, dat1, dat2, dat3,
             wb_w, wb_y, wb_x, sem):
    cid = lax.axis_index("c")
    sid = lax.axis_index("s")
    half0 = cid * HHW

    HB = CHUNK // 2
    baseA = sid * HB
    baseB = HHW + baseA

    pltpu.sync_copy(scale_hbm, scale_v)
    pltpu.sync_copy(fx_hbm.at[pl.ds(baseA, HB)], fx_v.at[pl.ds(0, HB)])
    pltpu.sync_copy(fx_hbm.at[pl.ds(baseB, HB)], fx_v.at[pl.ds(HB, HB)])
    pltpu.sync_copy(fy_hbm.at[pl.ds(baseA, HB)], fy_v.at[pl.ds(0, HB)])
    pltpu.sync_copy(fy_hbm.at[pl.ds(baseB, HB)], fy_v.at[pl.ds(HB, HB)])

    pltpu.sync_copy(zeros_hbm, wb_w)
    for acc in (acc_w, acc_y, acc_x):
        pltpu.sync_copy(wb_w.at[pl.ds(0, ZSL)], acc.at[pl.ds(sid * ZSL, ZSL)])
    plsc.subcore_barrier()

    sc = scale_v[...]
    lanes = lax.iota(jnp.int32, 16)
    fzero = jnp.zeros((16,), jnp.float32)
    fone = jnp.ones((16,), jnp.float32)

    rows_per_round = W // RPX
    rowsA = sid * (HB // W)
    NRH = NR // 2

    def round_body(r, carry):
        anys = [None, None, None, None]
        rr = jnp.where(r < NRH, r, r - NRH)
        row_in_block = rr // rows_per_round
        yrow = rowsA + row_in_block + jnp.where(r < NRH, 0, H // 2)
        x0 = (rr - row_in_block * rows_per_round) * RPX
        yrow_f = yrow.astype(jnp.float32)
        for g in range(NG):
            off = r * RPX + g * L
            q = off + lanes
            xi = x0 + g * L + lanes
            fyv = fy_v[pl.ds(off, L)]
            fxv = fx_v[pl.ds(off, L)]
            wy = yrow_f + sc * fyv
            wx = xi.astype(jnp.float32) + sc * fxv
            inb = ((wy >= 0.0) & (wy < float(H))
                   & (wx >= 0.0) & (wx < float(W)))
            inbf = jnp.where(inb, fone, fzero)
            wy = wy * inbf
            wx = wx * inbf
            t = wy.astype(jnp.int32)
            l = wx.astype(jnp.int32)
            dy = wy - t.astype(jnp.float32)
            dx = wx - l.astype(jnp.float32)
            bok = (t + 1) < H
            rok = (l + 1) < W
            bf = jnp.where(bok, fone, fzero)
            rf = jnp.where(rok, fone, fzero)
            omdy = 1.0 - dy
            omdx = 1.0 - dx
            w_tl = omdy * omdx * inbf
            w_tr = omdy * dx * rf * inbf
            w_bl = dy * omdx * bf * inbf
            w_br = dy * dx * rf * bf * inbf
            tl = t * W + l
            garbage = HHW + (q & (GPAD - 1))
            for ci, (idx_ref, dat_ref, ok_c, i_c, w_c) in enumerate((
                    (idx0, dat0, inb, tl, w_tl),
                    (idx1, dat1, inb & rok, tl + 1, w_tr),
                    (idx2, dat2, inb & bok, tl + W, w_bl),
                    (idx3, dat3, inb & rok & bok, tl + W + 1, w_br))):
                local = ok_c & (i_c >= half0) & (i_c < half0 + HHW)
                idx_ref[pl.ds(g * L, L)] = jnp.where(local, i_c - half0,
                                                     garbage)
                dat_ref[pl.ds(g * L, L)] = w_c
                dat_ref[pl.ds(RPX + g * L, L)] = w_c * fyv
                dat_ref[pl.ds(2 * RPX + g * L, L)] = w_c * fxv
                any_g = jnp.any(local)
                anys[ci] = any_g if g == 0 else (anys[ci] | any_g)

        corners = ((anys[0], idx0, dat0), (anys[1], idx1, dat1),
                   (anys[2], idx2, dat2), (anys[3], idx3, dat3))
        for any_c, idx_ref, dat_ref in corners:
            @pl.when(any_c)
            def _(idx_ref=idx_ref, dat_ref=dat_ref):
                pltpu.async_copy(dat_ref.at[pl.ds(0, RPX)],
                                 acc_w.at[idx_ref], sem, add=True)
                pltpu.async_copy(dat_ref.at[pl.ds(RPX, RPX)],
                                 acc_y.at[idx_ref], sem, add=True)
                pltpu.async_copy(dat_ref.at[pl.ds(2 * RPX, RPX)],
                                 acc_x.at[idx_ref], sem, add=True)

        for any_c, idx_ref, dat_ref in corners:
            @pl.when(any_c)
            def _(dat_ref=dat_ref):
                for ch in range(3):
                    pltpu.make_async_copy(
                        zeros_hbm.at[pl.ds(0, RPX)],
                        dat_ref.at[pl.ds(ch * RPX, RPX)], sem).wait()

        return carry

    lax.fori_loop(0, NR, round_body, 0)
    plsc.subcore_barrier()

    o = sid * OSL
    pltpu.sync_copy(acc_w.at[pl.ds(o, OSL)], wb_w.at[pl.ds(0, OSL)])
    pltpu.sync_copy(acc_y.at[pl.ds(o, OSL)], wb_y)
    pltpu.sync_copy(acc_x.at[pl.ds(o, OSL)], wb_x)

    def div_body(i, carry):
        s = pl.ds(i * L, L)
        wsum = wb_w[s] + 1e-9
        wb_y[s] = wb_y[s] / wsum
        wb_x[s] = wb_x[s] / wsum
        return carry

    lax.fori_loop(0, OSL // L, div_body, 0)
    gbase = cid * HHW + o
    pltpu.sync_copy(wb_y, oy_hbm.at[pl.ds(gbase, OSL)])
    pltpu.sync_copy(wb_x, ox_hbm.at[pl.ds(gbase, OSL)])


_mesh = plsc.VectorSubcoreMesh(core_axis_name="c", subcore_axis_name="s")
_params = pltpu.CompilerParams(needs_layout_passes=False,
                               use_tc_tiling_on_sc=False)

_sc_call = pl.kernel(
    _sc_body,
    out_type=(jax.ShapeDtypeStruct((HW,), jnp.float32),
              jax.ShapeDtypeStruct((HW,), jnp.float32)),
    mesh=_mesh,
    compiler_params=_params,
    scratch_types=[
        pltpu.VMEM_SHARED((ACC_ROWS,), jnp.float32),
        pltpu.VMEM_SHARED((ACC_ROWS,), jnp.float32),
        pltpu.VMEM_SHARED((ACC_ROWS,), jnp.float32),
        pltpu.VMEM((16,), jnp.float32),
        pltpu.VMEM((CHUNK,), jnp.float32),
        pltpu.VMEM((CHUNK,), jnp.float32),
        pltpu.VMEM((RPX,), jnp.int32),
        pltpu.VMEM((RPX,), jnp.int32),
        pltpu.VMEM((RPX,), jnp.int32),
        pltpu.VMEM((RPX,), jnp.int32),
        pltpu.VMEM((3 * RPX,), jnp.float32),
        pltpu.VMEM((3 * RPX,), jnp.float32),
        pltpu.VMEM((3 * RPX,), jnp.float32),
        pltpu.VMEM((3 * RPX,), jnp.float32),
        pltpu.VMEM((ZSL,), jnp.float32),
        pltpu.VMEM((OSL,), jnp.float32),
        pltpu.VMEM((OSL,), jnp.float32),
        pltpu.SemaphoreType.DMA,
    ],
)


@jax.jit
def kernel(i, tref, flow_maps_x, flow_maps_y):
    fx = lax.dynamic_index_in_dim(flow_maps_x[0], i, axis=0, keepdims=False)
    fy = lax.dynamic_index_in_dim(flow_maps_y[0], i, axis=0, keepdims=False)
    fx_flat = fx.reshape(HW)
    fy_flat = fy.reshape(HW)
    scale = jnp.asarray(tref - i, jnp.float32)
    scale_arr = jnp.broadcast_to(scale, (16,))
    zeros = jnp.zeros((ZSL,), jnp.float32)
    oy, ox = _sc_call(scale_arr, fx_flat, fy_flat, zeros)
    return (ox.reshape(1, 1, H, W), oy.reshape(1, 1, H, W))


# speedup vs baseline: 1.7632x; 1.3363x over previous
import jax
import jax.numpy as jnp
from jax import lax
from jax.experimental import pallas as pl
from jax.experimental.pallas import tpu as pltpu
from jax.experimental.pallas import tpu_sc as plsc

H, W = 480, 640
HW = H * W
HHW = HW // 2
NC, NS, L = 2, 16, 16
CHUNK = HW // NS
RPX = 128
NR = CHUNK // RPX
NG = RPX // L
GPAD = 2048
ACC_ROWS = HHW + GPAD
ZSL = ACC_ROWS // NS
OSL = HHW // NS


def _sc_body(scale_hbm, fx_hbm, fy_hbm, zeros_hbm, oy_hbm, ox_hbm,
             acc_w, acc_y, acc_x, scale_v, fx_v, fy_v,
             idx0, idx1, idx2, idx3, dat0, dat1, dat2, dat3,
             idx4, idx5, idx6, idx7, dat4, dat5, dat6, dat7,
             wb_w, wb_y, wb_x, sem):
    cid = lax.axis_index("c")
    sid = lax.axis_index("s")
    half0 = cid * HHW

    HB = CHUNK // 2
    baseA = sid * HB
    baseB = HHW + baseA

    pltpu.sync_copy(scale_hbm, scale_v)
    pltpu.sync_copy(fx_hbm.at[pl.ds(baseA, HB)], fx_v.at[pl.ds(0, HB)])
    pltpu.sync_copy(fx_hbm.at[pl.ds(baseB, HB)], fx_v.at[pl.ds(HB, HB)])
    pltpu.sync_copy(fy_hbm.at[pl.ds(baseA, HB)], fy_v.at[pl.ds(0, HB)])
    pltpu.sync_copy(fy_hbm.at[pl.ds(baseB, HB)], fy_v.at[pl.ds(HB, HB)])

    pltpu.sync_copy(zeros_hbm, wb_w)
    for acc in (acc_w, acc_y, acc_x):
        pltpu.sync_copy(wb_w.at[pl.ds(0, ZSL)], acc.at[pl.ds(sid * ZSL, ZSL)])
    plsc.subcore_barrier()

    sc = scale_v[...]
    lanes = lax.iota(jnp.int32, 16)
    fzero = jnp.zeros((16,), jnp.float32)
    fone = jnp.ones((16,), jnp.float32)

    rows_per_round = W // RPX
    rowsA = sid * (HB // W)
    NRH = NR // 2

    SETS = ((idx0, idx1, idx2, idx3, dat0, dat1, dat2, dat3),
            (idx4, idx5, idx6, idx7, dat4, dat5, dat6, dat7))

    def _drain(dat_ref):
        for ch in range(3):
            pltpu.make_async_copy(zeros_hbm.at[pl.ds(0, RPX)],
                                  dat_ref.at[pl.ds(ch * RPX, RPX)], sem).wait()

    def _subround(r, bufs, prev):
        ia, ib, ic, id_, da, db, dc, dd = bufs
        for c in range(4):
            @pl.when(prev[c])
            def _(dat_ref=bufs[4 + c]):
                _drain(dat_ref)
        anys = [None, None, None, None]
        rr = jnp.where(r < NRH, r, r - NRH)
        row_in_block = rr // rows_per_round
        yrow = rowsA + row_in_block + jnp.where(r < NRH, 0, H // 2)
        x0 = (rr - row_in_block * rows_per_round) * RPX
        yrow_f = yrow.astype(jnp.float32)
        for g in range(NG):
            off = r * RPX + g * L
            q = off + lanes
            xi = x0 + g * L + lanes
            fyv = fy_v[pl.ds(off, L)]
            fxv = fx_v[pl.ds(off, L)]
            wy = yrow_f + sc * fyv
            wx = xi.astype(jnp.float32) + sc * fxv
            inb = ((wy >= 0.0) & (wy < float(H))
                   & (wx >= 0.0) & (wx < float(W)))
            inbf = jnp.where(inb, fone, fzero)
            wy = wy * inbf
            wx = wx * inbf
            t = wy.astype(jnp.int32)
            l = wx.astype(jnp.int32)
            dy = wy - t.astype(jnp.float32)
            dx = wx - l.astype(jnp.float32)
            bok = (t + 1) < H
            rok = (l + 1) < W
            bf = jnp.where(bok, fone, fzero)
            rf = jnp.where(rok, fone, fzero)
            omdy = 1.0 - dy
            omdx = 1.0 - dx
            w_tl = omdy * omdx * inbf
            w_tr = omdy * dx * rf * inbf
            w_bl = dy * omdx * bf * inbf
            w_br = dy * dx * rf * bf * inbf
            tl = t * W + l
            garbage = HHW + (q & (GPAD - 1))
            for ci, (idx_ref, dat_ref, ok_c, i_c, w_c) in enumerate((
                    (ia, da, inb, tl, w_tl),
                    (ib, db, inb & rok, tl + 1, w_tr),
                    (ic, dc, inb & bok, tl + W, w_bl),
                    (id_, dd, inb & rok & bok, tl + W + 1, w_br))):
                local = ok_c & (i_c >= half0) & (i_c < half0 + HHW)
                idx_ref[pl.ds(g * L, L)] = jnp.where(local, i_c - half0,
                                                     garbage)
                dat_ref[pl.ds(g * L, L)] = w_c
                dat_ref[pl.ds(RPX + g * L, L)] = w_c * fyv
                dat_ref[pl.ds(2 * RPX + g * L, L)] = w_c * fxv
                any_g = jnp.any(local)
                anys[ci] = any_g if g == 0 else (anys[ci] | any_g)

        for c in range(4):
            @pl.when(anys[c])
            def _(idx_ref=bufs[c], dat_ref=bufs[4 + c]):
                pltpu.async_copy(dat_ref.at[pl.ds(0, RPX)],
                                 acc_w.at[idx_ref], sem, add=True)
                pltpu.async_copy(dat_ref.at[pl.ds(RPX, RPX)],
                                 acc_y.at[idx_ref], sem, add=True)
                pltpu.async_copy(dat_ref.at[pl.ds(2 * RPX, RPX)],
                                 acc_x.at[idx_ref], sem, add=True)
        return anys

    def pair_body(pp, carry):
        pA = carry[0:4]
        pB = carry[4:8]
        aA = _subround(2 * pp, SETS[0], pA)
        aB = _subround(2 * pp + 1, SETS[1], pB)
        return tuple(aA) + tuple(aB)

    f = jnp.zeros((), jnp.bool_)
    final = lax.fori_loop(0, NR // 2, pair_body, (f, f, f, f, f, f, f, f))
    for c in range(4):
        @pl.when(final[c])
        def _(dat_ref=SETS[0][4 + c]):
            _drain(dat_ref)
        @pl.when(final[4 + c])
        def _(dat_ref=SETS[1][4 + c]):
            _drain(dat_ref)

    plsc.subcore_barrier()

    o = sid * OSL
    pltpu.sync_copy(acc_w.at[pl.ds(o, OSL)], wb_w.at[pl.ds(0, OSL)])
    pltpu.sync_copy(acc_y.at[pl.ds(o, OSL)], wb_y)
    pltpu.sync_copy(acc_x.at[pl.ds(o, OSL)], wb_x)

    def div_body(i, carry):
        s = pl.ds(i * L, L)
        wsum = wb_w[s] + 1e-9
        wb_y[s] = wb_y[s] / wsum
        wb_x[s] = wb_x[s] / wsum
        return carry

    lax.fori_loop(0, OSL // L, div_body, 0)
    gbase = cid * HHW + o
    pltpu.sync_copy(wb_y, oy_hbm.at[pl.ds(gbase, OSL)])
    pltpu.sync_copy(wb_x, ox_hbm.at[pl.ds(gbase, OSL)])


_mesh = plsc.VectorSubcoreMesh(core_axis_name="c", subcore_axis_name="s")
_params = pltpu.CompilerParams(needs_layout_passes=False,
                               use_tc_tiling_on_sc=False)

_sc_call = pl.kernel(
    _sc_body,
    out_type=(jax.ShapeDtypeStruct((HW,), jnp.float32),
              jax.ShapeDtypeStruct((HW,), jnp.float32)),
    mesh=_mesh,
    compiler_params=_params,
    scratch_types=[
        pltpu.VMEM_SHARED((ACC_ROWS,), jnp.float32),
        pltpu.VMEM_SHARED((ACC_ROWS,), jnp.float32),
        pltpu.VMEM_SHARED((ACC_ROWS,), jnp.float32),
        pltpu.VMEM((16,), jnp.float32),
        pltpu.VMEM((CHUNK,), jnp.float32),
        pltpu.VMEM((CHUNK,), jnp.float32),
        pltpu.VMEM((RPX,), jnp.int32),
        pltpu.VMEM((RPX,), jnp.int32),
        pltpu.VMEM((RPX,), jnp.int32),
        pltpu.VMEM((RPX,), jnp.int32),
        pltpu.VMEM((3 * RPX,), jnp.float32),
        pltpu.VMEM((3 * RPX,), jnp.float32),
        pltpu.VMEM((3 * RPX,), jnp.float32),
        pltpu.VMEM((3 * RPX,), jnp.float32),
        pltpu.VMEM((RPX,), jnp.int32),
        pltpu.VMEM((RPX,), jnp.int32),
        pltpu.VMEM((RPX,), jnp.int32),
        pltpu.VMEM((RPX,), jnp.int32),
        pltpu.VMEM((3 * RPX,), jnp.float32),
        pltpu.VMEM((3 * RPX,), jnp.float32),
        pltpu.VMEM((3 * RPX,), jnp.float32),
        pltpu.VMEM((3 * RPX,), jnp.float32),
        pltpu.VMEM((ZSL,), jnp.float32),
        pltpu.VMEM((OSL,), jnp.float32),
        pltpu.VMEM((OSL,), jnp.float32),
        pltpu.SemaphoreType.DMA,
    ],
)


@jax.jit
def kernel(i, tref, flow_maps_x, flow_maps_y):
    fx = lax.dynamic_index_in_dim(flow_maps_x[0], i, axis=0, keepdims=False)
    fy = lax.dynamic_index_in_dim(flow_maps_y[0], i, axis=0, keepdims=False)
    fx_flat = fx.reshape(HW)
    fy_flat = fy.reshape(HW)
    scale = jnp.asarray(tref - i, jnp.float32)
    scale_arr = jnp.broadcast_to(scale, (16,))
    zeros = jnp.zeros((ZSL,), jnp.float32)
    oy, ox = _sc_call(scale_arr, fx_flat, fy_flat, zeros)
    return (ox.reshape(1, 1, H, W), oy.reshape(1, 1, H, W))


# speedup vs baseline: 1.8127x; 1.0281x over previous
import jax
import jax.numpy as jnp
from jax import lax
from jax.experimental import pallas as pl
from jax.experimental.pallas import tpu as pltpu
from jax.experimental.pallas import tpu_sc as plsc

H, W = 480, 640
HW = H * W
HHW = HW // 2
NC, NS, L = 2, 16, 16
CHUNK = HW // NS
RPX = 128
NR = CHUNK // RPX
NG = RPX // L
GPAD = 2048
ACC_ROWS = HHW + GPAD
ZSL = ACC_ROWS // NS
OSL = HHW // NS


def _sc_body(scale_hbm, fx_hbm, fy_hbm, zeros_hbm, oy_hbm, ox_hbm,
             acc_w, acc_y, acc_x, scale_v, fx_v, fy_v,
             idx0, idx1, idx2, idx3, dat0, dat1, dat2, dat3,
             idx4, idx5, idx6, idx7, dat4, dat5, dat6, dat7,
             wb_w, wb_y, wb_x, sem):
    cid = lax.axis_index("c")
    sid = lax.axis_index("s")
    half0 = cid * HHW

    HB = CHUNK // 2
    baseA = sid * HB
    baseB = HHW + baseA

    d = [pltpu.async_copy(scale_hbm, scale_v, sem),
         pltpu.async_copy(fx_hbm.at[pl.ds(baseA, HB)], fx_v.at[pl.ds(0, HB)],
                          sem),
         pltpu.async_copy(fx_hbm.at[pl.ds(baseB, HB)], fx_v.at[pl.ds(HB, HB)],
                          sem),
         pltpu.async_copy(fy_hbm.at[pl.ds(baseA, HB)], fy_v.at[pl.ds(0, HB)],
                          sem),
         pltpu.async_copy(fy_hbm.at[pl.ds(baseB, HB)], fy_v.at[pl.ds(HB, HB)],
                          sem),
         pltpu.async_copy(zeros_hbm, wb_w, sem)]
    for desc in d:
        desc.wait()

    dz = [pltpu.async_copy(wb_w.at[pl.ds(0, ZSL)],
                           acc.at[pl.ds(sid * ZSL, ZSL)], sem)
          for acc in (acc_w, acc_y, acc_x)]
    for desc in dz:
        desc.wait()
    plsc.subcore_barrier()

    sc = scale_v[...]
    lanes = lax.iota(jnp.int32, 16)
    fzero = jnp.zeros((16,), jnp.float32)
    fone = jnp.ones((16,), jnp.float32)

    rows_per_round = W // RPX
    rowsA = sid * (HB // W)
    NRH = NR // 2

    SETS = ((idx0, idx1, idx2, idx3, dat0, dat1, dat2, dat3),
            (idx4, idx5, idx6, idx7, dat4, dat5, dat6, dat7))

    def _drain(dat_ref):
        for ch in range(3):
            pltpu.make_async_copy(zeros_hbm.at[pl.ds(0, RPX)],
                                  dat_ref.at[pl.ds(ch * RPX, RPX)], sem).wait()

    def _subround(r, bufs, prev):
        ia, ib, ic, id_, da, db, dc, dd = bufs
        for c in range(4):
            @pl.when(prev[c])
            def _(dat_ref=bufs[4 + c]):
                _drain(dat_ref)
        anys = [None, None, None, None]
        rr = jnp.where(r < NRH, r, r - NRH)
        row_in_block = rr // rows_per_round
        yrow = rowsA + row_in_block + jnp.where(r < NRH, 0, H // 2)
        x0 = (rr - row_in_block * rows_per_round) * RPX
        yrow_f = yrow.astype(jnp.float32)
        for g in range(NG):
            off = r * RPX + g * L
            q = off + lanes
            xi = x0 + g * L + lanes
            fyv = fy_v[pl.ds(off, L)]
            fxv = fx_v[pl.ds(off, L)]
            wy = yrow_f + sc * fyv
            wx = xi.astype(jnp.float32) + sc * fxv
            inb = ((wy >= 0.0) & (wy < float(H))
                   & (wx >= 0.0) & (wx < float(W)))
            inbf = jnp.where(inb, fone, fzero)
            wy = wy * inbf
            wx = wx * inbf
            t = wy.astype(jnp.int32)
            l = wx.astype(jnp.int32)
            dy = wy - t.astype(jnp.float32)
            dx = wx - l.astype(jnp.float32)
            bok = (t + 1) < H
            rok = (l + 1) < W
            bf = jnp.where(bok, fone, fzero)
            rf = jnp.where(rok, fone, fzero)
            omdy = 1.0 - dy
            omdx = 1.0 - dx
            w_tl = omdy * omdx * inbf
            w_tr = omdy * dx * rf * inbf
            w_bl = dy * omdx * bf * inbf
            w_br = dy * dx * rf * bf * inbf
            tl = t * W + l
            garbage = HHW + (q & (GPAD - 1))
            for ci, (idx_ref, dat_ref, ok_c, i_c, w_c) in enumerate((
                    (ia, da, inb, tl, w_tl),
                    (ib, db, inb & rok, tl + 1, w_tr),
                    (ic, dc, inb & bok, tl + W, w_bl),
                    (id_, dd, inb & rok & bok, tl + W + 1, w_br))):
                local = ok_c & (i_c >= half0) & (i_c < half0 + HHW)
                idx_ref[pl.ds(g * L, L)] = jnp.where(local, i_c - half0,
                                                     garbage)
                dat_ref[pl.ds(g * L, L)] = w_c
                dat_ref[pl.ds(RPX + g * L, L)] = w_c * fyv
                dat_ref[pl.ds(2 * RPX + g * L, L)] = w_c * fxv
                any_g = jnp.any(local)
                anys[ci] = any_g if g == 0 else (anys[ci] | any_g)

        for c in range(4):
            @pl.when(anys[c])
            def _(idx_ref=bufs[c], dat_ref=bufs[4 + c]):
                pltpu.async_copy(dat_ref.at[pl.ds(0, RPX)],
                                 acc_w.at[idx_ref], sem, add=True)
                pltpu.async_copy(dat_ref.at[pl.ds(RPX, RPX)],
                                 acc_y.at[idx_ref], sem, add=True)
                pltpu.async_copy(dat_ref.at[pl.ds(2 * RPX, RPX)],
                                 acc_x.at[idx_ref], sem, add=True)
        return anys

    def pair_body(pp, carry):
        pA = carry[0:4]
        pB = carry[4:8]
        aA = _subround(2 * pp, SETS[0], pA)
        aB = _subround(2 * pp + 1, SETS[1], pB)
        return tuple(aA) + tuple(aB)

    f = jnp.zeros((), jnp.bool_)
    final = lax.fori_loop(0, NR // 2, pair_body, (f, f, f, f, f, f, f, f))
    for c in range(4):
        @pl.when(final[c])
        def _(dat_ref=SETS[0][4 + c]):
            _drain(dat_ref)
        @pl.when(final[4 + c])
        def _(dat_ref=SETS[1][4 + c]):
            _drain(dat_ref)

    plsc.subcore_barrier()

    o = sid * OSL
    dn = [pltpu.async_copy(acc_w.at[pl.ds(o, OSL)], wb_w.at[pl.ds(0, OSL)],
                           sem),
          pltpu.async_copy(acc_y.at[pl.ds(o, OSL)], wb_y, sem),
          pltpu.async_copy(acc_x.at[pl.ds(o, OSL)], wb_x, sem)]
    for desc in dn:
        desc.wait()

    def div_body(i, carry):
        s = pl.ds(i * L, L)
        wsum = wb_w[s] + 1e-9
        wb_y[s] = wb_y[s] / wsum
        wb_x[s] = wb_x[s] / wsum
        return carry

    lax.fori_loop(0, OSL // L, div_body, 0)
    gbase = cid * HHW + o
    pltpu.sync_copy(wb_y, oy_hbm.at[pl.ds(gbase, OSL)])
    pltpu.sync_copy(wb_x, ox_hbm.at[pl.ds(gbase, OSL)])


_mesh = plsc.VectorSubcoreMesh(core_axis_name="c", subcore_axis_name="s")
_params = pltpu.CompilerParams(needs_layout_passes=False,
                               use_tc_tiling_on_sc=False)

_sc_call = pl.kernel(
    _sc_body,
    out_type=(jax.ShapeDtypeStruct((HW,), jnp.float32),
              jax.ShapeDtypeStruct((HW,), jnp.float32)),
    mesh=_mesh,
    compiler_params=_params,
    scratch_types=[
        pltpu.VMEM_SHARED((ACC_ROWS,), jnp.float32),
        pltpu.VMEM_SHARED((ACC_ROWS,), jnp.float32),
        pltpu.VMEM_SHARED((ACC_ROWS,), jnp.float32),
        pltpu.VMEM((16,), jnp.float32),
        pltpu.VMEM((CHUNK,), jnp.float32),
        pltpu.VMEM((CHUNK,), jnp.float32),
        pltpu.VMEM((RPX,), jnp.int32),
        pltpu.VMEM((RPX,), jnp.int32),
        pltpu.VMEM((RPX,), jnp.int32),
        pltpu.VMEM((RPX,), jnp.int32),
        pltpu.VMEM((3 * RPX,), jnp.float32),
        pltpu.VMEM((3 * RPX,), jnp.float32),
        pltpu.VMEM((3 * RPX,), jnp.float32),
        pltpu.VMEM((3 * RPX,), jnp.float32),
        pltpu.VMEM((RPX,), jnp.int32),
        pltpu.VMEM((RPX,), jnp.int32),
        pltpu.VMEM((RPX,), jnp.int32),
        pltpu.VMEM((RPX,), jnp.int32),
        pltpu.VMEM((3 * RPX,), jnp.float32),
        pltpu.VMEM((3 * RPX,), jnp.float32),
        pltpu.VMEM((3 * RPX,), jnp.float32),
        pltpu.VMEM((3 * RPX,), jnp.float32),
        pltpu.VMEM((ZSL,), jnp.float32),
        pltpu.VMEM((OSL,), jnp.float32),
        pltpu.VMEM((OSL,), jnp.float32),
        pltpu.SemaphoreType.DMA,
    ],
)


@jax.jit
def kernel(i, tref, flow_maps_x, flow_maps_y):
    fx = lax.dynamic_index_in_dim(flow_maps_x[0], i, axis=0, keepdims=False)
    fy = lax.dynamic_index_in_dim(flow_maps_y[0], i, axis=0, keepdims=False)
    fx_flat = fx.reshape(HW)
    fy_flat = fy.reshape(HW)
    scale = jnp.asarray(tref - i, jnp.float32)
    scale_arr = jnp.broadcast_to(scale, (16,))
    zeros = jnp.zeros((ZSL,), jnp.float32)
    oy, ox = _sc_call(scale_arr, fx_flat, fy_flat, zeros)
    return (ox.reshape(1, 1, H, W), oy.reshape(1, 1, H, W))
